# baseline (speedup 1.0000x reference)
import functools

import jax
import jax.numpy as jnp
from jax import lax
from jax.experimental import pallas as pl
from jax.experimental.pallas import tpu as pltpu
from jax.experimental.pallas import tpu_sc as plsc

B, L, EMB, HID, TAGS = 4096, 200, 64, 256, 5
NC, NS = 2, 16
NW = NC * NS
EX_PER_W = B // NW
TOK_PER_W = EX_PER_W * L
CHUNK = 256
NCHUNK = TOK_PER_W // CHUNK
NBUF = 4


def _sc_pooled_sums(x3, seg3, emb_table):
    mesh = plsc.VectorSubcoreMesh(core_axis_name="c", subcore_axis_name="s")

    @functools.partial(
        pl.kernel,
        out_type=jax.ShapeDtypeStruct((B, EMB), jnp.float32),
        mesh=mesh,
        scratch_types=[
            pltpu.VMEM((NCHUNK, CHUNK), jnp.int32),
            pltpu.VMEM((NCHUNK, CHUNK), jnp.int32),
            *([pltpu.VMEM((CHUNK, EMB), jnp.float32)] * NBUF),
            pltpu.VMEM_SHARED((NS * EX_PER_W, EMB), jnp.float32),
            *([pltpu.SemaphoreType.DMA] * (2 * NBUF)),
        ],
        compiler_params=pltpu.CompilerParams(use_tc_tiling_on_sc=False),
    )
    def k(x_hbm, seg_hbm, table_hbm, out_hbm, x_v, seg_v, *rest):
        bufs = rest[:NBUF]
        acc_sh = rest[NBUF]
        gsem = rest[NBUF + 1:NBUF + 1 + NBUF]
        ssem = rest[NBUF + 1 + NBUF:]

        sid = lax.axis_index("s")
        w = sid * NC + lax.axis_index("c")
        pltpu.sync_copy(x_hbm.at[w], x_v)
        pltpu.sync_copy(seg_hbm.at[sid], seg_v)

        zeros = jnp.zeros((16,), jnp.float32)

        @pl.loop(0, EX_PER_W)
        def _(r):
            @pl.loop(0, EMB, step=16)
            def _(j):
                bufs[0].at[r, pl.ds(j, 16)][...] = zeros

        pltpu.sync_copy(bufs[0].at[pl.ds(0, EX_PER_W)],
                        acc_sh.at[pl.ds(sid * EX_PER_W, EX_PER_W)])

        for b in range(NBUF):
            pltpu.async_copy(table_hbm.at[x_v.at[b]], bufs[b], gsem[b])

        @pl.loop(0, NCHUNK, step=NBUF)
        def _(c0):
            scatters = []
            for b in range(NBUF):
                pltpu.make_async_copy(
                    table_hbm.at[x_v.at[0]], bufs[b], gsem[b]).wait()
                scatters.append(pltpu.async_copy(
                    bufs[b], acc_sh.at[seg_v.at[c0 + b]], ssem[b], add=True))
            for b in range(NBUF):
                scatters[b].wait()

                @pl.when(c0 + NBUF + b < NCHUNK)
                def _():
                    pltpu.async_copy(
                        table_hbm.at[x_v.at[c0 + NBUF + b]], bufs[b], gsem[b])

        pltpu.sync_copy(acc_sh.at[pl.ds(sid * EX_PER_W, EX_PER_W)],
                        out_hbm.at[pl.ds(w * EX_PER_W, EX_PER_W)])

    return k(x3, seg3, emb_table)


def _tc_head(sums, x, W1, b1, W2, b2, Wo_p, bo_p):
    blk = 512
    grid = (B // blk,)

    def body(sums_ref, x_ref, w1_ref, b1_ref, w2_ref, b2_ref, wo_ref, bo_ref,
             out_ref):
        cnt = jnp.sum((x_ref[...] != 0).astype(jnp.float32), axis=1,
                      keepdims=True)
        pooled = sums_ref[...] / jnp.maximum(cnt, 1.0)
        h = jnp.dot(pooled, w1_ref[...], preferred_element_type=jnp.float32,
                    precision=lax.Precision.HIGHEST) + b1_ref[...]
        h = jnp.maximum(h, 0.0)
        h = jnp.dot(h, w2_ref[...], preferred_element_type=jnp.float32,
                    precision=lax.Precision.HIGHEST) + b2_ref[...]
        h = jnp.maximum(h, 0.0)
        out_ref[...] = jnp.dot(h, wo_ref[...],
                               preferred_element_type=jnp.float32,
                               precision=lax.Precision.HIGHEST) + bo_ref[...]

    return pl.pallas_call(
        body,
        grid=grid,
        in_specs=[
            pl.BlockSpec((blk, EMB), lambda i: (i, 0)),
            pl.BlockSpec((blk, L), lambda i: (i, 0)),
            pl.BlockSpec((EMB, HID), lambda i: (0, 0)),
            pl.BlockSpec((1, HID), lambda i: (0, 0)),
            pl.BlockSpec((HID, HID), lambda i: (0, 0)),
            pl.BlockSpec((1, HID), lambda i: (0, 0)),
            pl.BlockSpec((HID, 128), lambda i: (0, 0)),
            pl.BlockSpec((1, 128), lambda i: (0, 0)),
        ],
        out_specs=pl.BlockSpec((blk, 128), lambda i: (i, 0)),
        out_shape=jax.ShapeDtypeStruct((B, 128), jnp.float32),
    )(sums, x, W1, b1, W2, b2, Wo_p, bo_p)


def kernel(x, emb_table, W1, b1, W2, b2, Wout, bout):
    x = x.astype(jnp.int32)
    x3 = x.reshape(NW, NCHUNK, CHUNK)
    seg2 = (jnp.arange(TOK_PER_W, dtype=jnp.int32) // L).reshape(NCHUNK, CHUNK)
    seg3 = seg2[None, :, :] + (
        jnp.arange(NS, dtype=jnp.int32) * EX_PER_W)[:, None, None]

    sums = _sc_pooled_sums(x3, seg3, emb_table)

    Wo_p = jnp.zeros((HID, 128), jnp.float32).at[:, :TAGS].set(Wout)
    bo_p = jnp.zeros((1, 128), jnp.float32).at[:, :TAGS].set(bout[None, :])
    scores_p = _tc_head(sums, x, W1, b1[None, :], W2, b2[None, :], Wo_p, bo_p)
    return scores_p[:, :TAGS]


# speedup vs baseline: 1.0447x; 1.0447x over previous
import functools

import jax
import jax.numpy as jnp
from jax import lax
from jax.experimental import pallas as pl
from jax.experimental.pallas import tpu as pltpu
from jax.experimental.pallas import tpu_sc as plsc

B, L, EMB, HID, TAGS = 4096, 200, 64, 256, 5
NC, NS = 2, 16
NW = NC * NS
EX_PER_W = B // NW
TOK_PER_W = EX_PER_W * L
CHUNK = 128
NCHUNK = TOK_PER_W // CHUNK
NBUF = 8


def _sc_pooled_sums(x3, seg3, emb_table):
    mesh = plsc.VectorSubcoreMesh(core_axis_name="c", subcore_axis_name="s")

    @functools.partial(
        pl.kernel,
        out_type=jax.ShapeDtypeStruct((B, EMB), jnp.float32),
        mesh=mesh,
        scratch_types=[
            pltpu.VMEM((NCHUNK, CHUNK), jnp.int32),
            pltpu.VMEM((NCHUNK, CHUNK), jnp.int32),
            *([pltpu.VMEM((CHUNK, EMB), jnp.float32)] * NBUF),
            pltpu.VMEM_SHARED((NS * EX_PER_W, EMB), jnp.float32),
            *([pltpu.SemaphoreType.DMA] * (2 * NBUF)),
        ],
        compiler_params=pltpu.CompilerParams(use_tc_tiling_on_sc=False),
    )
    def k(x_hbm, seg_hbm, table_hbm, out_hbm, x_v, seg_v, *rest):
        bufs = rest[:NBUF]
        acc_sh = rest[NBUF]
        gsem = rest[NBUF + 1:NBUF + 1 + NBUF]
        ssem = rest[NBUF + 1 + NBUF:]

        sid = lax.axis_index("s")
        w = sid * NC + lax.axis_index("c")
        pltpu.sync_copy(x_hbm.at[w], x_v)
        pltpu.sync_copy(seg_hbm.at[sid], seg_v)

        zeros = jnp.zeros((16,), jnp.float32)

        @pl.loop(0, CHUNK)
        def _(r):
            @pl.loop(0, EMB, step=16)
            def _(j):
                bufs[0].at[r, pl.ds(j, 16)][...] = zeros

        pltpu.sync_copy(bufs[0], acc_sh.at[pl.ds(sid * EX_PER_W, EX_PER_W)])

        for b in range(NBUF):
            pltpu.async_copy(table_hbm.at[x_v.at[b]], bufs[b], gsem[b])

        @pl.loop(0, NCHUNK, step=NBUF)
        def _(c0):
            scatters = []
            for b in range(NBUF):
                pltpu.make_async_copy(
                    table_hbm.at[x_v.at[0]], bufs[b], gsem[b]).wait()
                scatters.append(pltpu.async_copy(
                    bufs[b], acc_sh.at[seg_v.at[c0 + b]], ssem[b], add=True))
            for b in range(NBUF):
                scatters[b].wait()

                @pl.when(c0 + NBUF + b < NCHUNK)
                def _():
                    pltpu.async_copy(
                        table_hbm.at[x_v.at[c0 + NBUF + b]], bufs[b], gsem[b])

        pltpu.sync_copy(acc_sh.at[pl.ds(sid * EX_PER_W, EX_PER_W)],
                        out_hbm.at[pl.ds(w * EX_PER_W, EX_PER_W)])

    return k(x3, seg3, emb_table)


def _tc_head(sums, x, W1, b1, W2, b2, Wo_p, bo_p):
    blk = 512
    grid = (B // blk,)

    def body(sums_ref, x_ref, w1_ref, b1_ref, w2_ref, b2_ref, wo_ref, bo_ref,
             out_ref):
        cnt = jnp.sum((x_ref[...] != 0).astype(jnp.float32), axis=1,
                      keepdims=True)
        pooled = sums_ref[...] / jnp.maximum(cnt, 1.0)
        h = jnp.dot(pooled, w1_ref[...], preferred_element_type=jnp.float32,
                    precision=lax.Precision.HIGHEST) + b1_ref[...]
        h = jnp.maximum(h, 0.0)
        h = jnp.dot(h, w2_ref[...], preferred_element_type=jnp.float32,
                    precision=lax.Precision.HIGHEST) + b2_ref[...]
        h = jnp.maximum(h, 0.0)
        out_ref[...] = jnp.dot(h, wo_ref[...],
                               preferred_element_type=jnp.float32,
                               precision=lax.Precision.HIGHEST) + bo_ref[...]

    return pl.pallas_call(
        body,
        grid=grid,
        in_specs=[
            pl.BlockSpec((blk, EMB), lambda i: (i, 0)),
            pl.BlockSpec((blk, L), lambda i: (i, 0)),
            pl.BlockSpec((EMB, HID), lambda i: (0, 0)),
            pl.BlockSpec((1, HID), lambda i: (0, 0)),
            pl.BlockSpec((HID, HID), lambda i: (0, 0)),
            pl.BlockSpec((1, HID), lambda i: (0, 0)),
            pl.BlockSpec((HID, 128), lambda i: (0, 0)),
            pl.BlockSpec((1, 128), lambda i: (0, 0)),
        ],
        out_specs=pl.BlockSpec((blk, 128), lambda i: (i, 0)),
        out_shape=jax.ShapeDtypeStruct((B, 128), jnp.float32),
    )(sums, x, W1, b1, W2, b2, Wo_p, bo_p)


def kernel(x, emb_table, W1, b1, W2, b2, Wout, bout):
    x = x.astype(jnp.int32)
    x3 = x.reshape(NW, NCHUNK, CHUNK)
    seg2 = (jnp.arange(TOK_PER_W, dtype=jnp.int32) // L).reshape(NCHUNK, CHUNK)
    seg3 = seg2[None, :, :] + (
        jnp.arange(NS, dtype=jnp.int32) * EX_PER_W)[:, None, None]

    sums = _sc_pooled_sums(x3, seg3, emb_table)

    Wo_p = jnp.zeros((HID, 128), jnp.float32).at[:, :TAGS].set(Wout)
    bo_p = jnp.zeros((1, 128), jnp.float32).at[:, :TAGS].set(bout[None, :])
    scores_p = _tc_head(sums, x, W1, b1[None, :], W2, b2[None, :], Wo_p, bo_p)
    return scores_p[:, :TAGS]
